# baseline (speedup 1.0000x reference)
import functools

import jax
import jax.numpy as jnp
from jax import lax
from jax.experimental import pallas as pl
from jax.experimental.pallas import tpu as pltpu
from jax.experimental.pallas import tpu_sc as plsc

BEAM = 8
VOCAB = 1000000
LANES = 16
NWORKERS = 32
PER_WORKER = VOCAB * BEAM // NWORKERS
NCHUNK = 5
CHUNK = PER_WORKER // NCHUNK
VREGS_PER_CHUNK = CHUNK // LANES

NEG_INF = float("-inf")


def _sc_scan_body(x, m_out, s_out, v_out, i_out, buf, fstage, istage, sem0, sem1):
    wid = lax.axis_index("c") * 16 + lax.axis_index("s")
    rel_base = (wid % 4) * PER_WORKER

    lanes = lax.iota(jnp.int32, LANES)

    m = jnp.full((LANES,), NEG_INF, jnp.float32)
    s = jnp.zeros((LANES,), jnp.float32)
    tv = jnp.full((LANES,), NEG_INF, jnp.float32)
    ti = jnp.zeros((LANES,), jnp.int32)
    th = jnp.full((LANES,), NEG_INF, jnp.float32)

    sems = (sem0, sem1)
    cps = [None, None]
    cps[0] = pltpu.async_copy(x.at[wid, 0], buf.at[0], sems[0])
    for k in range(NCHUNK):
        cur = k % 2
        if k + 1 < NCHUNK:
            nxt = (k + 1) % 2
            cps[nxt] = pltpu.async_copy(x.at[wid, k + 1], buf.at[nxt], sems[nxt])
        cps[cur].wait()
        chunk_base = rel_base + k * CHUNK

        def body(i, carry, cur=cur, chunk_base=chunk_base):
            m, s, tv, ti, th = carry
            v = buf[cur, pl.ds(i * LANES, LANES)]
            exceed = jnp.any((v > m) | (v > th))

            def slow(_):
                nm = jnp.maximum(m, v)
                ns = s * jnp.exp(m - nm) + jnp.exp(v - nm)
                gi = (chunk_base + i * LANES) + lanes
                sv, si = plsc.sort_key_val(v, gi, descending=True)
                rv = lax.rev(sv, (0,))
                ri = lax.rev(si, (0,))
                take = tv >= rv
                mv = jnp.maximum(tv, rv)
                mi = jnp.where(take, ti, ri)
                ntv, nti = plsc.sort_key_val(mv, mi, descending=True)
                nth = jnp.broadcast_to(jnp.min(ntv), (LANES,))
                return nm, ns, ntv, nti, nth

            def fast(_):
                return m, s + jnp.exp(v - m), tv, ti, th

            return lax.cond(exceed, slow, fast, None)

        m, s, tv, ti, th = lax.fori_loop(
            0, VREGS_PER_CHUNK, body, (m, s, tv, ti, th))

    fstage[...] = m
    pltpu.sync_copy(fstage, m_out.at[wid])
    fstage[...] = s
    pltpu.sync_copy(fstage, s_out.at[wid])
    fstage[...] = tv
    pltpu.sync_copy(fstage, v_out.at[wid])
    istage[...] = ti
    pltpu.sync_copy(istage, i_out.at[wid])


def _sc_scan(x3):
    mesh = plsc.VectorSubcoreMesh(core_axis_name="c", subcore_axis_name="s")
    f = pl.kernel(
        _sc_scan_body,
        mesh=mesh,
        out_type=[
            jax.ShapeDtypeStruct((NWORKERS, LANES), jnp.float32),
            jax.ShapeDtypeStruct((NWORKERS, LANES), jnp.float32),
            jax.ShapeDtypeStruct((NWORKERS, LANES), jnp.float32),
            jax.ShapeDtypeStruct((NWORKERS, LANES), jnp.int32),
        ],
        scratch_types=[
            pltpu.VMEM((2, CHUNK), jnp.float32),
            pltpu.VMEM((LANES,), jnp.float32),
            pltpu.VMEM((LANES,), jnp.int32),
            pltpu.SemaphoreType.DMA,
            pltpu.SemaphoreType.DMA,
        ],
        compiler_params=pltpu.CompilerParams(
            use_tc_tiling_on_sc=False, needs_layout_passes=False),
    )
    return f(x3)


def _merge_body(step_ref, m_ref, s_ref, tv_ref, ti_ref, sc_ref, gs_ref,
                ns_out, gq_out):
    step = step_ref[0, 0]
    m = m_ref[...]
    s = s_ref[...]
    tv = tv_ref[...]
    ti = ti_ref[...]

    rows = lax.broadcasted_iota(jnp.int32, (NWORKERS, LANES), 0)
    beam_of_row = rows // 4

    adj = jnp.zeros((NWORKERS, LANES), jnp.float32)
    for b in range(BEAM):
        mb = m[4 * b:4 * b + 4, :]
        sb = s[4 * b:4 * b + 4, :]
        mx = jnp.max(mb)
        tot = jnp.sum(sb * jnp.exp(mb - mx))
        lse = mx + jnp.log(tot)
        adj_b = sc_ref[0, b] - lse
        adj = jnp.where(beam_of_row == b, adj_b, adj)

    a = tv + adj
    lin = rows * LANES + lax.broadcasted_iota(jnp.int32, (NWORKERS, LANES), 1)
    big = jnp.int32(1 << 30)

    vals, toks, beams = [], [], []
    for _ in range(BEAM):
        mx = jnp.max(a)
        msk = a == mx
        loc = jnp.min(jnp.where(msk, lin, big))
        pick = lin == loc
        toks.append(jnp.max(jnp.where(pick, ti, -1)))
        beams.append(jnp.max(jnp.where(pick, beam_of_row, -1)))
        vals.append(mx)
        a = jnp.where(pick, NEG_INF, a)

    ns_out[...] = jnp.concatenate(
        [v.reshape(1, 1) for v in vals], axis=1)

    g_rows = []
    for j in range(BEAM):
        acc = gs_ref[0, :]
        for b in range(1, BEAM):
            acc = jnp.where(beams[j] == b, gs_ref[b, :], acc)
        g_rows.append(acc.reshape(1, 256))
    g = jnp.concatenate(g_rows, axis=0)

    rowidx = lax.broadcasted_iota(jnp.int32, (BEAM, 256), 0)
    colidx = lax.broadcasted_iota(jnp.int32, (BEAM, 256), 1)
    tokmat = jnp.zeros((BEAM, 256), jnp.int32)
    for j in range(BEAM):
        tokmat = jnp.where(rowidx == j, toks[j], tokmat)
    gq_out[...] = jnp.where(colidx == step, tokmat, g)


def _merge(step_arr, m, s, tv, ti, scores2, gen_seq):
    return pl.pallas_call(
        _merge_body,
        in_specs=[
            pl.BlockSpec(memory_space=pltpu.SMEM),
            pl.BlockSpec(memory_space=pltpu.VMEM),
            pl.BlockSpec(memory_space=pltpu.VMEM),
            pl.BlockSpec(memory_space=pltpu.VMEM),
            pl.BlockSpec(memory_space=pltpu.VMEM),
            pl.BlockSpec(memory_space=pltpu.VMEM),
            pl.BlockSpec(memory_space=pltpu.VMEM),
        ],
        out_specs=[
            pl.BlockSpec(memory_space=pltpu.VMEM),
            pl.BlockSpec(memory_space=pltpu.VMEM),
        ],
        out_shape=[
            jax.ShapeDtypeStruct((1, BEAM), jnp.float32),
            jax.ShapeDtypeStruct((BEAM, 256), jnp.int32),
        ],
    )(step_arr, m, s, tv, ti, scores2, gen_seq)


def kernel(dec_output, scores, gen_seq, step):
    x = dec_output[:, -1, :].reshape(NWORKERS, NCHUNK, CHUNK)
    step_arr = jnp.asarray(step, jnp.int32).reshape(1, 1)
    m, s, tv, ti = _sc_scan(x)
    ns, gq = _merge(step_arr, m, s, tv, ti,
                    scores.reshape(1, BEAM), gen_seq)
    return ns.reshape(BEAM), gq
